# baseline (speedup 1.0000x reference)
import functools

import jax
import jax.numpy as jnp
from jax import lax
from jax.experimental import pallas as pl
from jax.experimental.pallas import tpu as pltpu
from jax.experimental.pallas import tpu_sc as plsc

N = 320000
D = 128
OUT = 128
R = 1280000

NC = 2
NS = 16
NW = NC * NS

SEG_WIN = 320
NWIN = N // SEG_WIN
WPW = (NWIN + NW - 1) // NW
E = 128
ACC_ROWS = SEG_WIN + 8
ZR = 64
PK = OUT // 2


def _mm_body(x_ref, w_ref, hb_ref, o_ref):
    xw = jnp.dot(x_ref[...], w_ref[...], preferred_element_type=jnp.float32)
    o_ref[...] = xw + hb_ref[...]


def _matmul_bias(X, W, half_bias):
    BN = 2000
    return pl.pallas_call(
        _mm_body,
        grid=(N // BN,),
        in_specs=[pl.BlockSpec((BN, D), lambda i: (i, 0)),
                  pl.BlockSpec((D, OUT), lambda i: (0, 0)),
                  pl.BlockSpec((1, OUT), lambda i: (0, 0))],
        out_specs=pl.BlockSpec((BN, OUT), lambda i: (i, 0)),
        out_shape=jax.ShapeDtypeStruct((N, OUT), jnp.float32),
    )(X, W, half_bias.reshape(1, OUT))


def _final_body(x_ref, conv_ref, w_ref, wb_ref, b_ref, o_ref):
    xw = jnp.dot(x_ref[...], w_ref[...], preferred_element_type=jnp.float32)
    xwb = jnp.dot(x_ref[...], wb_ref[...], preferred_element_type=jnp.float32)
    o_ref[...] = jnp.maximum(xw + xwb * conv_ref[...] + b_ref[...], 0.0)


def _final(X, conv, W, W_back, b):
    BN = 2000
    return pl.pallas_call(
        _final_body,
        grid=(N // BN,),
        in_specs=[pl.BlockSpec((BN, D), lambda i: (i, 0)),
                  pl.BlockSpec((BN, OUT), lambda i: (i, 0)),
                  pl.BlockSpec((D, OUT), lambda i: (0, 0)),
                  pl.BlockSpec((D, OUT), lambda i: (0, 0)),
                  pl.BlockSpec((1, OUT), lambda i: (0, 0))],
        out_specs=pl.BlockSpec((BN, OUT), lambda i: (i, 0)),
        out_shape=jax.ShapeDtypeStruct((N, OUT), jnp.float32),
    )(X, conv, W, W_back, b.reshape(1, OUT))


_SC_MESH = plsc.VectorSubcoreMesh(core_axis_name="c", subcore_axis_name="s",
                                  num_cores=NC, num_subcores=NS)


@functools.partial(
    pl.kernel,
    out_type=jax.ShapeDtypeStruct((N, OUT), jnp.float32),
    mesh=_SC_MESH,
    compiler_params=pltpu.CompilerParams(needs_layout_passes=False),
    scratch_types=[
        pltpu.VMEM((2, E), jnp.int32),
        pltpu.VMEM((2, E), jnp.int32),
        pltpu.VMEM((E,), jnp.int32),
        pltpu.VMEM((E,), jnp.int32),
        pltpu.VMEM((E,), jnp.int32),
        pltpu.VMEM((E,), jnp.int32),
        pltpu.VMEM((E, OUT), jnp.float32),
        pltpu.VMEM((E, OUT), jnp.float32),
        pltpu.VMEM((E, OUT), jnp.float32),
        pltpu.VMEM((E, OUT), jnp.float32),
        pltpu.VMEM((ZR, OUT), jnp.float32),
        pltpu.VMEM((16,), jnp.int32),
        pltpu.VMEM_SHARED((NS * ACC_ROWS, OUT), jnp.float32),
        pltpu.SemaphoreType.DMA,
        pltpu.SemaphoreType.DMA,
        pltpu.SemaphoreType.DMA,
        pltpu.SemaphoreType.DMA,
        pltpu.SemaphoreType.DMA,
        pltpu.SemaphoreType.DMA,
        pltpu.SemaphoreType.DMA,
        pltpu.SemaphoreType.DMA,
        pltpu.SemaphoreType.DMA,
        pltpu.SemaphoreType.DMA,
    ],
)
def _sc_conv(xwp_hbm, eab_hbm, ebk_hbm, wtab_hbm, conv_hbm,
             iab0, iab1, bk0, bk1, li0, li1, ra0, ra1, rb0, rb1,
             zbuf, wrow, acc,
             semI0, semI1, semK0, semK1, semA0, semA1, semB0, semB1,
             semS0, semS1):
    c = lax.axis_index("c")
    s = lax.axis_index("s")
    wid = s * NC + c
    base = s * ACC_ROWS
    iab = (iab0, iab1)
    bk = (bk0, bk1)
    li = (li0, li1)
    ra = (ra0, ra1)
    rb = (rb0, rb1)
    semI = (semI0, semI1)
    semK = (semK0, semK1)
    semA = (semA0, semA1)
    semB = (semB0, semB1)
    semS = (semS0, semS1)

    @pl.loop(0, ZR)
    def _zero(r):
        for t in range(OUT // 16):
            zbuf[r, pl.ds(t * 16, 16)] = jnp.zeros((16,), jnp.float32)

    @pl.loop(0, WPW)
    def _win(j):
        i = wid + NW * j

        @pl.when(i < NWIN)
        def _():
            pltpu.sync_copy(wtab_hbm.at[i], wrow)
            v = wrow[...]
            c0 = v[0]
            nch = v[8]
            seg0 = i * SEG_WIN

            def idx_start(k, p):
                pltpu.make_async_copy(
                    eab_hbm.at[c0 + k], iab[p], semI[p]).start()
                pltpu.make_async_copy(
                    ebk_hbm.at[c0 + k], bk[p], semK[p]).start()

            def idx_wait(p):
                pltpu.make_async_copy(
                    eab_hbm.at[0], iab[p], semI[p]).wait()
                pltpu.make_async_copy(
                    ebk_hbm.at[0], bk[p], semK[p]).wait()

            def gather_a(p):
                return pltpu.make_async_copy(
                    xwp_hbm.at[iab[p].at[0]], ra[p], semA[p])

            def gather_b(p):
                return pltpu.make_async_copy(
                    xwp_hbm.at[iab[p].at[1]], rb[p], semB[p])

            def scatter_start(p):
                pltpu.async_copy(ra[p], acc.at[li[p]], semS[p], add=True)

            def scatter_wait(p):
                pltpu.make_async_copy(ra[p], acc.at[li[p]], semS[p]).wait()

            for q in range(SEG_WIN // ZR):
                pltpu.sync_copy(zbuf, acc.at[pl.ds(base + q * ZR, ZR)])

            @pl.when(nch > 0)
            def _():
                idx_start(0, 0)

            @pl.when(nch > 1)
            def _():
                idx_start(1, 1)

            @pl.when(nch > 0)
            def _():
                idx_wait(0)
                gather_a(0).start()
                gather_b(0).start()

            @pl.loop(0, (nch + 1) // 2)
            def _pair(t):
                for p in range(2):
                    k = 2 * t + p
                    np_ = 1 - p

                    @pl.when(k < nch)
                    def _():
                        @pl.when(k + 1 < nch)
                        def _():
                            @pl.when(k >= 1)
                            def _():
                                scatter_wait(np_)

                            idx_wait(np_)
                            gather_a(np_).start()
                            gather_b(np_).start()

                        gather_a(p).wait()
                        gather_b(p).wait()

                        @pl.when(k + 2 < nch)
                        def _():
                            idx_start(k + 2, p)

                        for g in range(E // 16):
                            sl = pl.ds(g * 16, 16)
                            bkv = bk[p][sl]
                            l = bkv - seg0
                            inwin = (l >= 0) & (l < SEG_WIN)
                            li[p][sl] = jnp.where(inwin, l, SEG_WIN) + base

                        @pl.loop(0, E, unroll=4)
                        def _row(r):
                            for tt in range(OUT // 16):
                                sl = pl.ds(tt * 16, 16)
                                ra[p][r, sl] = jnp.maximum(
                                    ra[p][r, sl] + rb[p][r, sl], 0.0)

                        scatter_start(p)

            @pl.when(nch > 0)
            def _():
                scatter_wait(0)

            @pl.when(nch > 1)
            def _():
                scatter_wait(1)

            pltpu.sync_copy(acc.at[pl.ds(base, SEG_WIN)],
                            conv_hbm.at[pl.ds(seg0, SEG_WIN)])


def _window_table(backref):
    bounds = jnp.searchsorted(
        backref, jnp.arange(0, N + 1, SEG_WIN, dtype=jnp.int32)
    ).astype(jnp.int32)
    c0 = bounds[:-1] // E
    c1 = (bounds[1:] + E - 1) // E
    nch = c1 - c0
    return jnp.concatenate(
        [jnp.broadcast_to(c0[:, None], (NWIN, 8)),
         jnp.broadcast_to(nch[:, None], (NWIN, 8))], axis=1)


def kernel(X, ref_a, ref_b, backref, e_map, v_count, W, W_back, W_prop, b,
           b_prop):
    xwp = _matmul_bias(X, W_prop, 0.5 * b_prop)
    wtab = _window_table(backref)
    eab = jnp.stack([ref_a.reshape(R // E, E), ref_b.reshape(R // E, E)],
                    axis=1)
    ebk = backref.reshape(R // E, E)
    conv = _sc_conv(xwp, eab, ebk, wtab)
    x_out = _final(X, conv, W, W_back, b)
    return (x_out, ref_a, ref_b, backref, e_map, v_count)


# speedup vs baseline: 1.7988x; 1.7988x over previous
import functools

import jax
import jax.numpy as jnp
from jax import lax
from jax.experimental import pallas as pl
from jax.experimental.pallas import tpu as pltpu
from jax.experimental.pallas import tpu_sc as plsc

N = 320000
D = 128
OUT = 128
R = 1280000

NC = 2
NS = 16
NW = NC * NS

SEG_WIN = 320
NWIN = N // SEG_WIN
WPW = (NWIN + NW - 1) // NW
E = 128
ACC_ROWS = SEG_WIN + 8
ZR = 64
PK = OUT // 2


def _mm_body(x_ref, w_ref, hb_ref, o_ref):
    xw = jnp.dot(x_ref[...], w_ref[...], preferred_element_type=jnp.float32)
    o_ref[...] = xw + hb_ref[...]


def _matmul_bias(X, W, half_bias):
    BN = 2000
    return pl.pallas_call(
        _mm_body,
        grid=(N // BN,),
        in_specs=[pl.BlockSpec((BN, D), lambda i: (i, 0)),
                  pl.BlockSpec((D, OUT), lambda i: (0, 0)),
                  pl.BlockSpec((1, OUT), lambda i: (0, 0))],
        out_specs=pl.BlockSpec((BN, OUT), lambda i: (i, 0)),
        out_shape=jax.ShapeDtypeStruct((N, OUT), jnp.float32),
    )(X, W, half_bias.reshape(1, OUT))


def _final_body(x_ref, conv_ref, w_ref, wb_ref, b_ref, o_ref):
    xw = jnp.dot(x_ref[...], w_ref[...], preferred_element_type=jnp.float32)
    xwb = jnp.dot(x_ref[...], wb_ref[...], preferred_element_type=jnp.float32)
    o_ref[...] = jnp.maximum(xw + xwb * conv_ref[...] + b_ref[...], 0.0)


def _final(X, conv, W, W_back, b):
    BN = 2000
    return pl.pallas_call(
        _final_body,
        grid=(N // BN,),
        in_specs=[pl.BlockSpec((BN, D), lambda i: (i, 0)),
                  pl.BlockSpec((BN, OUT), lambda i: (i, 0)),
                  pl.BlockSpec((D, OUT), lambda i: (0, 0)),
                  pl.BlockSpec((D, OUT), lambda i: (0, 0)),
                  pl.BlockSpec((1, OUT), lambda i: (0, 0))],
        out_specs=pl.BlockSpec((BN, OUT), lambda i: (i, 0)),
        out_shape=jax.ShapeDtypeStruct((N, OUT), jnp.float32),
    )(X, conv, W, W_back, b.reshape(1, OUT))


_SC_MESH = plsc.VectorSubcoreMesh(core_axis_name="c", subcore_axis_name="s",
                                  num_cores=NC, num_subcores=NS)


@functools.partial(
    pl.kernel,
    out_type=jax.ShapeDtypeStruct((N, OUT), jnp.float32),
    mesh=_SC_MESH,
    compiler_params=pltpu.CompilerParams(needs_layout_passes=False),
    scratch_types=[
        pltpu.VMEM((2, E), jnp.int32),
        pltpu.VMEM((2, E), jnp.int32),
        pltpu.VMEM((E,), jnp.int32),
        pltpu.VMEM((E,), jnp.int32),
        pltpu.VMEM((E,), jnp.int32),
        pltpu.VMEM((E,), jnp.int32),
        pltpu.VMEM((E, OUT), jnp.float32),
        pltpu.VMEM((E, OUT), jnp.float32),
        pltpu.VMEM((E, OUT), jnp.float32),
        pltpu.VMEM((E, OUT), jnp.float32),
        pltpu.VMEM((ZR, OUT), jnp.float32),
        pltpu.VMEM((16,), jnp.int32),
        pltpu.VMEM_SHARED((NS * ACC_ROWS, OUT), jnp.float32),
        pltpu.SemaphoreType.DMA,
        pltpu.SemaphoreType.DMA,
        pltpu.SemaphoreType.DMA,
        pltpu.SemaphoreType.DMA,
        pltpu.SemaphoreType.DMA,
        pltpu.SemaphoreType.DMA,
        pltpu.SemaphoreType.DMA,
        pltpu.SemaphoreType.DMA,
        pltpu.SemaphoreType.DMA,
        pltpu.SemaphoreType.DMA,
    ],
)
def _sc_conv(xwp_hbm, eab_hbm, ebk_hbm, wtab_hbm, conv_hbm,
             iab0, iab1, bk0, bk1, li0, li1, ra0, ra1, rb0, rb1,
             zbuf, wrow, acc,
             semI0, semI1, semK0, semK1, semA0, semA1, semB0, semB1,
             semS0, semS1):
    c = lax.axis_index("c")
    s = lax.axis_index("s")
    wid = s * NC + c
    base = s * ACC_ROWS
    iab = (iab0, iab1)
    bk = (bk0, bk1)
    li = (li0, li1)
    ra = (ra0, ra1)
    rb = (rb0, rb1)
    semI = (semI0, semI1)
    semK = (semK0, semK1)
    semA = (semA0, semA1)
    semB = (semB0, semB1)
    semS = (semS0, semS1)

    @pl.loop(0, ZR)
    def _zero(r):
        for t in range(OUT // 16):
            zbuf[r, pl.ds(t * 16, 16)] = jnp.zeros((16,), jnp.float32)

    @pl.loop(0, WPW)
    def _win(j):
        i = wid + NW * j

        @pl.when(i < NWIN)
        def _():
            pltpu.sync_copy(wtab_hbm.at[i], wrow)
            v = wrow[...]
            c0 = v[0]
            nch = v[8]
            seg0 = i * SEG_WIN

            def idx_start(k, p):
                pltpu.make_async_copy(
                    eab_hbm.at[c0 + k], iab[p], semI[p]).start()
                pltpu.make_async_copy(
                    ebk_hbm.at[c0 + k], bk[p], semK[p]).start()

            def idx_wait(p):
                pltpu.make_async_copy(
                    eab_hbm.at[0], iab[p], semI[p]).wait()
                pltpu.make_async_copy(
                    ebk_hbm.at[0], bk[p], semK[p]).wait()

            def gather_a(p):
                return pltpu.make_async_copy(
                    xwp_hbm.at[iab[p].at[0]], ra[p], semA[p])

            def gather_b(p):
                return pltpu.make_async_copy(
                    xwp_hbm.at[iab[p].at[1]], rb[p], semB[p])

            def scatter_start(p):
                pltpu.async_copy(ra[p], acc.at[li[p]], semS[p], add=True)

            def scatter_wait(p):
                pltpu.make_async_copy(ra[p], acc.at[li[p]], semS[p]).wait()

            for q in range(SEG_WIN // ZR):
                pltpu.sync_copy(zbuf, acc.at[pl.ds(base + q * ZR, ZR)])

            @pl.when(nch > 0)
            def _():
                idx_start(0, 0)

            @pl.when(nch > 1)
            def _():
                idx_start(1, 1)

            @pl.when(nch > 0)
            def _():
                idx_wait(0)
                gather_a(0).start()
                gather_b(0).start()

            @pl.loop(0, (nch + 1) // 2)
            def _pair(t):
                for p in range(2):
                    k = 2 * t + p
                    np_ = 1 - p

                    @pl.when(k < nch)
                    def _():
                        @pl.when(k + 1 < nch)
                        def _():
                            @pl.when(k >= 1)
                            def _():
                                scatter_wait(np_)

                            idx_wait(np_)
                            gather_a(np_).start()
                            gather_b(np_).start()

                        gather_a(p).wait()
                        gather_b(p).wait()

                        @pl.when(k + 2 < nch)
                        def _():
                            idx_start(k + 2, p)

                        for g in range(E // 16):
                            sl = pl.ds(g * 16, 16)
                            bkv = bk[p][sl]
                            l = bkv - seg0
                            inwin = (l >= 0) & (l < SEG_WIN)
                            li[p][sl] = jnp.where(inwin, l, SEG_WIN) + base

                        @functools.partial(
                            plsc.parallel_loop, 0, E, unroll=8)
                        def _row(r):
                            for tt in range(OUT // 16):
                                sl = pl.ds(tt * 16, 16)
                                ra[p][r, sl] = jnp.maximum(
                                    ra[p][r, sl] + rb[p][r, sl], 0.0)

                        scatter_start(p)

            @pl.when(nch > 0)
            def _():
                scatter_wait(0)

            @pl.when(nch > 1)
            def _():
                scatter_wait(1)

            pltpu.sync_copy(acc.at[pl.ds(base, SEG_WIN)],
                            conv_hbm.at[pl.ds(seg0, SEG_WIN)])


def _window_table(backref):
    bounds = jnp.searchsorted(
        backref, jnp.arange(0, N + 1, SEG_WIN, dtype=jnp.int32)
    ).astype(jnp.int32)
    c0 = bounds[:-1] // E
    c1 = (bounds[1:] + E - 1) // E
    nch = c1 - c0
    return jnp.concatenate(
        [jnp.broadcast_to(c0[:, None], (NWIN, 8)),
         jnp.broadcast_to(nch[:, None], (NWIN, 8))], axis=1)


def kernel(X, ref_a, ref_b, backref, e_map, v_count, W, W_back, W_prop, b,
           b_prop):
    xwp = _matmul_bias(X, W_prop, 0.5 * b_prop)
    wtab = _window_table(backref)
    eab = jnp.stack([ref_a.reshape(R // E, E), ref_b.reshape(R // E, E)],
                    axis=1)
    ebk = backref.reshape(R // E, E)
    conv = _sc_conv(xwp, eab, ebk, wtab)
    x_out = _final(X, conv, W, W_back, b)
    return (x_out, ref_a, ref_b, backref, e_map, v_count)


# speedup vs baseline: 1.9565x; 1.0876x over previous
import functools

import jax
import jax.numpy as jnp
from jax import lax
from jax.experimental import pallas as pl
from jax.experimental.pallas import tpu as pltpu
from jax.experimental.pallas import tpu_sc as plsc

N = 320000
D = 128
OUT = 128
R = 1280000

NC = 2
NS = 16
NW = NC * NS

SEG_WIN = 400
NWIN = N // SEG_WIN
WPW = (NWIN + NW - 1) // NW
E = 128
ACC_ROWS = SEG_WIN + 8


def _mm_body(x_ref, w_ref, hb_ref, o_ref):
    xw = jnp.dot(x_ref[...], w_ref[...], preferred_element_type=jnp.float32)
    o_ref[...] = xw + hb_ref[...]


def _matmul_bias(X, W, half_bias):
    BN = 2000
    return pl.pallas_call(
        _mm_body,
        grid=(N // BN,),
        in_specs=[pl.BlockSpec((BN, D), lambda i: (i, 0)),
                  pl.BlockSpec((D, OUT), lambda i: (0, 0)),
                  pl.BlockSpec((1, OUT), lambda i: (0, 0))],
        out_specs=pl.BlockSpec((BN, OUT), lambda i: (i, 0)),
        out_shape=jax.ShapeDtypeStruct((N, OUT), jnp.float32),
    )(X, W, half_bias.reshape(1, OUT))


def _final_body(x_ref, conv_ref, w_ref, wb_ref, b_ref, o_ref):
    xw = jnp.dot(x_ref[...], w_ref[...], preferred_element_type=jnp.float32)
    xwb = jnp.dot(x_ref[...], wb_ref[...], preferred_element_type=jnp.float32)
    o_ref[...] = jnp.maximum(xw + xwb * conv_ref[...] + b_ref[...], 0.0)


def _final(X, conv, W, W_back, b):
    BN = 2000
    return pl.pallas_call(
        _final_body,
        grid=(N // BN,),
        in_specs=[pl.BlockSpec((BN, D), lambda i: (i, 0)),
                  pl.BlockSpec((BN, OUT), lambda i: (i, 0)),
                  pl.BlockSpec((D, OUT), lambda i: (0, 0)),
                  pl.BlockSpec((D, OUT), lambda i: (0, 0)),
                  pl.BlockSpec((1, OUT), lambda i: (0, 0))],
        out_specs=pl.BlockSpec((BN, OUT), lambda i: (i, 0)),
        out_shape=jax.ShapeDtypeStruct((N, OUT), jnp.float32),
    )(X, conv, W, W_back, b.reshape(1, OUT))


_SC_MESH = plsc.VectorSubcoreMesh(core_axis_name="c", subcore_axis_name="s",
                                  num_cores=NC, num_subcores=NS)


@functools.partial(
    pl.kernel,
    out_type=jax.ShapeDtypeStruct((N, OUT), jnp.float32),
    mesh=_SC_MESH,
    compiler_params=pltpu.CompilerParams(needs_layout_passes=False),
    scratch_types=[
        pltpu.VMEM((2, E), jnp.int32),
        pltpu.VMEM((2, E), jnp.int32),
        pltpu.VMEM((E,), jnp.int32),
        pltpu.VMEM((E,), jnp.int32),
        pltpu.VMEM((E, OUT), jnp.float32),
        pltpu.VMEM((E, OUT), jnp.float32),
        pltpu.VMEM((E, OUT), jnp.float32),
        pltpu.VMEM((E, OUT), jnp.float32),
        pltpu.VMEM((ACC_ROWS, OUT), jnp.float32),
        pltpu.VMEM((E,), jnp.int32),
        pltpu.VMEM((16,), jnp.int32),
        pltpu.SemaphoreType.DMA,
        pltpu.SemaphoreType.DMA,
        pltpu.SemaphoreType.DMA,
        pltpu.SemaphoreType.DMA,
        pltpu.SemaphoreType.DMA,
        pltpu.SemaphoreType.DMA,
        pltpu.SemaphoreType.DMA,
        pltpu.SemaphoreType.DMA,
    ],
)
def _sc_conv(xwp_hbm, eab_hbm, ebk_hbm, wtab_hbm, conv_hbm,
             iab0, iab1, bk0, bk1, ra0, ra1, rb0, rb1, acc, liq, wrow,
             semI0, semI1, semK0, semK1, semA0, semA1, semB0, semB1):
    c = lax.axis_index("c")
    s = lax.axis_index("s")
    wid = s * NC + c
    iab = (iab0, iab1)
    bk = (bk0, bk1)
    ra = (ra0, ra1)
    rb = (rb0, rb1)
    semI = (semI0, semI1)
    semK = (semK0, semK1)
    semA = (semA0, semA1)
    semB = (semB0, semB1)
    iota = lax.iota(jnp.int32, 16)

    @pl.loop(0, WPW)
    def _win(j):
        i = wid + NW * j

        @pl.when(i < NWIN)
        def _():
            pltpu.sync_copy(wtab_hbm.at[i], wrow)
            v = wrow[...]
            c0 = v[0]
            nch = v[8]
            seg0 = i * SEG_WIN

            def idx_start(k, p):
                pltpu.make_async_copy(
                    eab_hbm.at[c0 + k], iab[p], semI[p]).start()
                pltpu.make_async_copy(
                    ebk_hbm.at[c0 + k], bk[p], semK[p]).start()

            def idx_wait(p):
                pltpu.make_async_copy(
                    eab_hbm.at[0], iab[p], semI[p]).wait()
                pltpu.make_async_copy(
                    ebk_hbm.at[0], bk[p], semK[p]).wait()

            def gather_a(p):
                return pltpu.make_async_copy(
                    xwp_hbm.at[iab[p].at[0]], ra[p], semA[p])

            def gather_b(p):
                return pltpu.make_async_copy(
                    xwp_hbm.at[iab[p].at[1]], rb[p], semB[p])

            @pl.loop(0, SEG_WIN)
            def _zero(r):
                for t in range(OUT // 16):
                    acc[r, pl.ds(t * 16, 16)] = jnp.zeros((16,), jnp.float32)

            @pl.when(nch > 0)
            def _():
                idx_start(0, 0)

            @pl.when(nch > 1)
            def _():
                idx_start(1, 1)

            @pl.when(nch > 0)
            def _():
                idx_wait(0)
                gather_a(0).start()
                gather_b(0).start()

            @pl.loop(0, (nch + 1) // 2)
            def _pair(t):
                for p in range(2):
                    k = 2 * t + p
                    np_ = 1 - p

                    @pl.when(k < nch)
                    def _():
                        @pl.when(k + 1 < nch)
                        def _():
                            idx_wait(np_)
                            gather_a(np_).start()
                            gather_b(np_).start()

                        gather_a(p).wait()
                        gather_b(p).wait()

                        for g in range(E // 16):
                            sl = pl.ds(g * 16, 16)
                            l = bk[p][sl] - seg0
                            inwin = (l >= 0) & (l < SEG_WIN)
                            liq[sl] = jnp.where(inwin, l, SEG_WIN)

                        @pl.when(k + 2 < nch)
                        def _():
                            idx_start(k + 2, p)

                        @pl.loop(0, E // 16)
                        def _g(g):
                            sl = pl.ds(g * 16, 16)
                            liv = liq[sl]
                            for jj in range(16):
                                r = g * 16 + jj
                                rowv = lax.broadcast(liv[jj], (16,))
                                avs = [ra[p][r, pl.ds(t * 16, 16)]
                                       for t in range(OUT // 16)]
                                bvs = [rb[p][r, pl.ds(t * 16, 16)]
                                       for t in range(OUT // 16)]
                                svs = [jnp.maximum(a + b, 0.0)
                                       for a, b in zip(avs, bvs)]
                                for t in range(OUT // 16):
                                    plsc.addupdate_scatter(
                                        acc, [rowv, iota + t * 16], svs[t])

            pltpu.sync_copy(acc.at[pl.ds(0, SEG_WIN)],
                            conv_hbm.at[pl.ds(seg0, SEG_WIN)])


def _window_table(backref):
    bounds = jnp.searchsorted(
        backref, jnp.arange(0, N + 1, SEG_WIN, dtype=jnp.int32)
    ).astype(jnp.int32)
    c0 = bounds[:-1] // E
    c1 = (bounds[1:] + E - 1) // E
    nch = c1 - c0
    return jnp.concatenate(
        [jnp.broadcast_to(c0[:, None], (NWIN, 8)),
         jnp.broadcast_to(nch[:, None], (NWIN, 8))], axis=1)


def kernel(X, ref_a, ref_b, backref, e_map, v_count, W, W_back, W_prop, b,
           b_prop):
    xwp = _matmul_bias(X, W_prop, 0.5 * b_prop)
    wtab = _window_table(backref)
    eab = jnp.stack([ref_a.reshape(R // E, E), ref_b.reshape(R // E, E)],
                    axis=1)
    ebk = backref.reshape(R // E, E)
    conv = _sc_conv(xwp, eab, ebk, wtab)
    x_out = _final(X, conv, W, W_back, b)
    return (x_out, ref_a, ref_b, backref, e_map, v_count)
